# baseline (speedup 1.0000x reference)
import jax
import jax.numpy as jnp
from jax import lax
from jax.experimental import pallas as pl
from jax.experimental.pallas import tpu as pltpu
from jax.experimental.pallas import tpu_sc as plsc

SEQ = 8192
D_MODEL = 1024
NUM_CORES = 2
NUM_SUBCORES = 16
NUM_WORKERS = NUM_CORES * NUM_SUBCORES
ROWS_PER_WORKER = SEQ // NUM_WORKERS
CHUNK_ROWS = 32
NUM_CHUNKS = ROWS_PER_WORKER // CHUNK_ROWS


NBUF = 3


def _copy_body(pe_hbm, out_hbm, buf0, buf1, buf2, isem0, isem1, isem2,
               osem0, osem1, osem2):
    wid = lax.axis_index("s") * NUM_CORES + lax.axis_index("c")
    base = wid * ROWS_PER_WORKER
    bufs = (buf0, buf1, buf2)
    isems = (isem0, isem1, isem2)
    osems = (osem0, osem1, osem2)

    def start_in(j):
        b = j % NBUF
        return pltpu.async_copy(
            pe_hbm.at[pl.ds(base + j * CHUNK_ROWS, CHUNK_ROWS)], bufs[b], isems[b])

    in_h = {}
    out_h = {}
    in_h[0] = start_in(0)
    in_h[1] = start_in(1)
    for j in range(NUM_CHUNKS):
        b = j % NBUF
        in_h[j].wait()
        out_h[j] = pltpu.async_copy(
            bufs[b], out_hbm.at[pl.ds(base + j * CHUNK_ROWS, CHUNK_ROWS)], osems[b])
        if j + 2 < NUM_CHUNKS:
            if j >= 1:
                out_h[j - 1].wait()
            in_h[j + 2] = start_in(j + 2)
    out_h[NUM_CHUNKS - 2].wait()
    out_h[NUM_CHUNKS - 1].wait()


@jax.jit
def kernel(x, position_embeddings):
    mesh = plsc.VectorSubcoreMesh(core_axis_name="c", subcore_axis_name="s")
    out = pl.kernel(
        _copy_body,
        mesh=mesh,
        out_type=jax.ShapeDtypeStruct((SEQ, D_MODEL), jnp.float32),
        scratch_types=(
            [pltpu.VMEM((CHUNK_ROWS, D_MODEL), jnp.float32)] * NBUF
            + [pltpu.SemaphoreType.DMA] * (2 * NBUF)
        ),
    )(position_embeddings)
    return out[None]


# speedup vs baseline: 1.0389x; 1.0389x over previous
import jax
import jax.numpy as jnp
from jax import lax
from jax.experimental import pallas as pl
from jax.experimental.pallas import tpu as pltpu
from jax.experimental.pallas import tpu_sc as plsc

SEQ = 8192
D_MODEL = 1024
NUM_CORES = 2
NUM_SUBCORES = 16
NUM_WORKERS = NUM_CORES * NUM_SUBCORES
ROWS_PER_WORKER = SEQ // NUM_WORKERS
BUF_ROWS = 56
_CHUNKS = [(i * BUF_ROWS, BUF_ROWS) for i in range(ROWS_PER_WORKER // BUF_ROWS)]
_REM = ROWS_PER_WORKER - len(_CHUNKS) * BUF_ROWS
if _REM:
    _CHUNKS.append((len(_CHUNKS) * BUF_ROWS, _REM))
NUM_CHUNKS = len(_CHUNKS)


def _copy_body(pe_hbm, out_hbm, buf0, buf1, isem0, isem1, osem0, osem1):
    wid = lax.axis_index("s") * NUM_CORES + lax.axis_index("c")
    base = wid * ROWS_PER_WORKER
    bufs = (buf0, buf1)
    isems = (isem0, isem1)
    osems = (osem0, osem1)

    def start_in(j):
        off, n = _CHUNKS[j]
        b = j % 2
        return pltpu.async_copy(
            pe_hbm.at[pl.ds(base + off, n)], bufs[b].at[pl.ds(0, n)], isems[b])

    in_h = {}
    out_h = {}
    in_h[0] = start_in(0)
    for j in range(NUM_CHUNKS):
        off, n = _CHUNKS[j]
        b = j % 2
        if j >= 1:
            out_h[j - 1].wait()
        if j + 1 < NUM_CHUNKS:
            in_h[j + 1] = start_in(j + 1)
        in_h[j].wait()
        out_h[j] = pltpu.async_copy(
            bufs[b].at[pl.ds(0, n)], out_hbm.at[pl.ds(base + off, n)], osems[b])
    out_h[NUM_CHUNKS - 1].wait()


@jax.jit
def kernel(x, position_embeddings):
    mesh = plsc.VectorSubcoreMesh(core_axis_name="c", subcore_axis_name="s")
    out = pl.kernel(
        _copy_body,
        mesh=mesh,
        out_type=jax.ShapeDtypeStruct((SEQ, D_MODEL), jnp.float32),
        scratch_types=(
            [pltpu.VMEM((BUF_ROWS, D_MODEL), jnp.float32)] * 2
            + [pltpu.SemaphoreType.DMA] * 4
        ),
    )(position_embeddings)
    return out[None]
